# baseline (speedup 1.0000x reference)
import functools

import jax
import jax.numpy as jnp
from jax import lax
from jax.experimental import pallas as pl
from jax.experimental.pallas import tpu as pltpu
from jax.experimental.pallas import tpu_sc as plsc

B, T, D = 4, 2048, 768
V, K = 100000, 100
EPS = 1e-12

NC, NS = 2, 16
NW = NC * NS
N_TOK = B * T

H = 2
N_TOK_H = N_TOK // H
ROWS_PER_W = N_TOK_H // NW
CHUNK = 64
NCHUNK = ROWS_PER_W // CHUNK
NGRP = D // 32


@functools.lru_cache(maxsize=1)
def _get_sc_gather():
    mesh = plsc.VectorSubcoreMesh(
        core_axis_name="c", subcore_axis_name="s",
        num_cores=NC, num_subcores=NS,
    )

    @functools.partial(
        pl.kernel,
        out_type=jax.ShapeDtypeStruct((N_TOK_H, D // 2), jnp.int32),
        mesh=mesh,
        compiler_params=pltpu.CompilerParams(needs_layout_passes=False),
        scratch_types=[
            pltpu.VMEM((NCHUNK, CHUNK), jnp.int32),
            pltpu.VMEM((2, CHUNK, D), jnp.float32),
            pltpu.VMEM((CHUNK, D // 2), jnp.int32),
            pltpu.SemaphoreType.DMA,
            pltpu.SemaphoreType.DMA,
        ],
    )
    def _sc_gather(tok_hbm, idx_hbm, out_hbm, idx_v, rows_v, obuf, sem0, sem1):
        wid = lax.axis_index("s") * NC + lax.axis_index("c")
        base = wid * ROWS_PER_W
        sems = (sem0, sem1)
        pltpu.sync_copy(idx_hbm.at[wid], idx_v)
        copies = [None] * NCHUNK
        copies[0] = pltpu.async_copy(
            tok_hbm.at[idx_v.at[0]], rows_v.at[0], sems[0])
        for c in range(NCHUNK):
            if c + 1 < NCHUNK:
                nb = (c + 1) % 2
                copies[c + 1] = pltpu.async_copy(
                    tok_hbm.at[idx_v.at[c + 1]], rows_v.at[nb], sems[nb]
                )
            copies[c].wait()
            buf = c % 2
            rbuf = rows_v.at[buf]
            evens = lax.iota(jnp.int32, 16) * 2

            def to_bf16(r, carry):
                rv = jnp.full((16,), r, jnp.int32)
                for g in range(NGRP):
                    ia = evens + (g * 32)
                    a = plsc.load_gather(rbuf, [rv, ia])
                    b = plsc.load_gather(rbuf, [rv, ia + 1])
                    au = lax.bitcast_convert_type(a, jnp.uint32)
                    bu = lax.bitcast_convert_type(b, jnp.uint32)
                    w = ((au + jnp.uint32(0x8000)) >> 16) | (
                        (bu + jnp.uint32(0x8000)) & jnp.uint32(0xFFFF0000))
                    obuf[r, pl.ds(g * 16, 16)] = lax.bitcast_convert_type(
                        w, jnp.int32)
                return carry

            lax.fori_loop(0, CHUNK, to_bf16, 0)
            pltpu.sync_copy(
                obuf, out_hbm.at[pl.ds(base + c * CHUNK, CHUNK)]
            )

    return _sc_gather


BT = 1024
_T_BLKS = T // BT
_B_H = B // H


def _tc_body_first(gath_ref, p_ref, s_ref, lang_ref, sw_ref, pos_ref,
                   g_ref, b_ref, out_ref):
    x = gath_ref[...].astype(jnp.float32) + pos_ref[...]
    x = x + jnp.dot(p_ref[...], lang_ref[...],
                    preferred_element_type=jnp.float32)
    x = x + s_ref[...] * sw_ref[...]
    mu = jnp.mean(x, axis=1, keepdims=True)
    xc = x - mu
    var = jnp.mean(xc * xc, axis=1, keepdims=True)
    out_ref[...] = xc * lax.rsqrt(var + EPS) * g_ref[...] + b_ref[...]


def _tc_body_rest(gath_ref, p_ref, s_ref, lang_ref, sw_ref, pos_ref,
                  g_ref, b_ref, prev_ref, out_ref):
    del prev_ref
    _tc_body_first(gath_ref, p_ref, s_ref, lang_ref, sw_ref, pos_ref,
                   g_ref, b_ref, out_ref)


def _make_tc(h):
    off = h * _B_H * _T_BLKS

    def full(i, j):
        return (off + j * _T_BLKS + i, 0)

    def half(i, j):
        return (j * _T_BLKS + i, 0)

    in_specs = [
        pl.BlockSpec((BT, D), half),
        pl.BlockSpec((BT, K), full),
        pl.BlockSpec((BT, 1), full),
        pl.BlockSpec((K, D), lambda i, j: (0, 0)),
        pl.BlockSpec((1, D), lambda i, j: (0, 0)),
        pl.BlockSpec((BT, D), lambda i, j: (i, 0)),
        pl.BlockSpec((1, D), lambda i, j: (0, 0)),
        pl.BlockSpec((1, D), lambda i, j: (0, 0)),
    ]
    kwargs = {}
    body = _tc_body_first
    if h > 0:
        in_specs.append(pl.BlockSpec(memory_space=pl.ANY))
        kwargs["input_output_aliases"] = {8: 0}
        body = _tc_body_rest
    return pl.pallas_call(
        body,
        grid=(_T_BLKS, _B_H),
        in_specs=in_specs,
        out_specs=pl.BlockSpec((BT, D), full),
        out_shape=jax.ShapeDtypeStruct((N_TOK, D), jnp.float32),
        **kwargs,
    )


def kernel(input_ids, p, s, tok_table, pos_table, lang_table, switch_emb,
           ln_gamma, ln_beta):
    ids = input_ids.astype(jnp.int32).reshape(H, NW, NCHUNK, CHUNK)
    p2 = p.reshape(N_TOK, K)
    s2 = s.reshape(N_TOK, 1)
    sw = switch_emb[None, :]
    g2 = ln_gamma[None, :]
    b2 = ln_beta[None, :]
    sc = _get_sc_gather()
    out = None
    for h in range(H):
        packed = sc(tok_table, ids[h])
        gath = jax.lax.bitcast_convert_type(
            packed, jnp.bfloat16).reshape(N_TOK_H, D)
        args = [gath, p2, s2, lang_table, sw, pos_table, g2, b2]
        if h > 0:
            args.append(out)
        out = _make_tc(h)(*args)
    return out.reshape(B, T, D)


# speedup vs baseline: 3.3074x; 3.3074x over previous
import functools

import jax
import jax.numpy as jnp
from jax import lax
from jax.experimental import pallas as pl
from jax.experimental.pallas import tpu as pltpu
from jax.experimental.pallas import tpu_sc as plsc

B, T, D = 4, 2048, 768
V, K = 100000, 100
EPS = 1e-12

NC, NS = 2, 16
NW = NC * NS
N_TOK = B * T

H = 2
T_H = T // H
N_TOK_H = N_TOK // H
ROWS_PER_W = N_TOK_H // NW
CHUNK = 64
NCHUNK = ROWS_PER_W // CHUNK


@functools.lru_cache(maxsize=1)
def _get_sc_gather():
    mesh = plsc.VectorSubcoreMesh(
        core_axis_name="c", subcore_axis_name="s",
        num_cores=NC, num_subcores=NS,
    )

    @functools.partial(
        pl.kernel,
        out_type=jax.ShapeDtypeStruct((N_TOK_H, D), jnp.float32),
        mesh=mesh,
        scratch_types=[
            pltpu.VMEM((NCHUNK, CHUNK), jnp.int32),
            pltpu.VMEM((2, CHUNK, D), jnp.float32),
            pltpu.SemaphoreType.DMA,
            pltpu.SemaphoreType.DMA,
        ],
    )
    def _sc_gather(tok_hbm, idx_hbm, out_hbm, idx_v, rows_v, sem0, sem1):
        wid = lax.axis_index("s") * NC + lax.axis_index("c")
        base = wid * ROWS_PER_W
        sems = (sem0, sem1)
        pltpu.sync_copy(idx_hbm.at[wid], idx_v)
        copies = [None] * NCHUNK
        copies[0] = pltpu.async_copy(
            tok_hbm.at[idx_v.at[0]], rows_v.at[0], sems[0])
        for c in range(NCHUNK):
            if c + 1 < NCHUNK:
                nb = (c + 1) % 2
                copies[c + 1] = pltpu.async_copy(
                    tok_hbm.at[idx_v.at[c + 1]], rows_v.at[nb], sems[nb]
                )
            copies[c].wait()
            pltpu.sync_copy(
                rows_v.at[c % 2], out_hbm.at[pl.ds(base + c * CHUNK, CHUNK)]
            )

    return _sc_gather


BT = 1024


def _tc_body_first(gath_ref, p_ref, s_ref, lang_ref, sw_ref, pos_ref,
                   g_ref, b_ref, out_ref):
    x = gath_ref[...] + pos_ref[...]
    x = x + jnp.dot(p_ref[...], lang_ref[...],
                    preferred_element_type=jnp.float32)
    x = x + s_ref[...] * sw_ref[...]
    mu = jnp.mean(x, axis=1, keepdims=True)
    xc = x - mu
    var = jnp.mean(xc * xc, axis=1, keepdims=True)
    out_ref[...] = xc * lax.rsqrt(var + EPS) * g_ref[...] + b_ref[...]


def _tc_body_rest(gath_ref, p_ref, s_ref, lang_ref, sw_ref, pos_ref,
                  g_ref, b_ref, prev_ref, out_ref):
    del prev_ref
    _tc_body_first(gath_ref, p_ref, s_ref, lang_ref, sw_ref, pos_ref,
                   g_ref, b_ref, out_ref)


def _make_tc(h):

    def full(j):
        return (j * H + h, 0)

    def half(j):
        return (j, 0)

    def fixed(j):
        return (0, 0)

    in_specs = [
        pl.BlockSpec((BT, D), half),
        pl.BlockSpec((BT, K), full),
        pl.BlockSpec((BT, 1), full),
        pl.BlockSpec((K, D), fixed),
        pl.BlockSpec((1, D), fixed),
        pl.BlockSpec((BT, D), lambda j: (h, 0)),
        pl.BlockSpec((1, D), fixed),
        pl.BlockSpec((1, D), fixed),
    ]
    kwargs = {}
    body = _tc_body_first
    if h > 0:
        in_specs.append(pl.BlockSpec(memory_space=pl.ANY))
        kwargs["input_output_aliases"] = {8: 0}
        body = _tc_body_rest
    return pl.pallas_call(
        body,
        grid=(B,),
        in_specs=in_specs,
        out_specs=pl.BlockSpec((BT, D), full),
        out_shape=jax.ShapeDtypeStruct((N_TOK, D), jnp.float32),
        **kwargs,
    )


def kernel(input_ids, p, s, tok_table, pos_table, lang_table, switch_emb,
           ln_gamma, ln_beta):
    ids = input_ids.astype(jnp.int32)
    p2 = p.reshape(N_TOK, K)
    s2 = s.reshape(N_TOK, 1)
    sw = switch_emb[None, :]
    g2 = ln_gamma[None, :]
    b2 = ln_beta[None, :]
    sc = _get_sc_gather()
    out = None
    for h in range(H):
        ids_h = ids[:, h * T_H:(h + 1) * T_H].reshape(NW, NCHUNK, CHUNK)
        gath = sc(tok_table, ids_h)
        args = [gath, p2, s2, lang_table, sw, pos_table, g2, b2]
        if h > 0:
            args.append(out)
        out = _make_tc(h)(*args)
    return out.reshape(B, T, D)
